# Initial kernel scaffold; baseline (speedup 1.0000x reference)
import jax
import jax.numpy as jnp
from jax.experimental import pallas as pl


def kernel(x, edge_index, edge_weight, batch, W1, b1, W2, b2, bn_gamma, bn_beta, bn_mean, bn_var, fc1_w, fc1_b, fc2_w, fc2_b):
    raise NotImplementedError("write your pallas kernel here")


# speedup vs baseline: 8.7791x; 8.7791x over previous
import functools

import jax
import jax.numpy as jnp
from jax import lax
from jax.experimental import pallas as pl
from jax.experimental.pallas import tpu as pltpu
from jax.experimental.pallas import tpu_sc as plsc

N_NODES = 100000
N_EDGES = 1600000
N_GRAPHS = 1024
D = 32

NC = 2
NS = 16

HALF = N_NODES // NC
CHB = 256
NCHUNK_B = 196
TRASH = NCHUNK_B * CHB
ACC_ROWS = TRASH + 256
ZPT = ACC_ROWS // NS
EC = 512
NCHUNK_A = N_EDGES // EC
KA = (NCHUNK_A + NS - 1) // NS
KB = (NCHUNK_B + NS - 1) // NS + 1
POOL_ROWS = N_GRAPHS + 256
BPAD = NCHUNK_B * CHB


def _sc_mesh():
    return plsc.VectorSubcoreMesh(core_axis_name="c", subcore_axis_name="s")


def _make_sc_layer(pool: bool):
    scratch = [
        pltpu.VMEM((4, 128), jnp.int32),
        pltpu.VMEM((4, 128), jnp.int32),
        pltpu.VMEM((EC,), jnp.float32),
        pltpu.VMEM((EC, D), jnp.float32),
        pltpu.VMEM_SHARED((ACC_ROWS, D), jnp.float32),
        pltpu.SemaphoreType.DMA,
    ]
    if pool:
        scratch += [
            pltpu.VMEM((CHB, D), jnp.float32),
            pltpu.VMEM((2, 128), jnp.int32),
            pltpu.VMEM((D,), jnp.float32),
            pltpu.VMEM((D,), jnp.float32),
            pltpu.VMEM_SHARED((POOL_ROWS, D), jnp.float32),
        ]
        out_type = jax.ShapeDtypeStruct((NC, N_GRAPHS, D), jnp.float32)
    else:
        out_type = jax.ShapeDtypeStruct((N_NODES, D), jnp.float32)

    def body(h_hbm, src_hbm, dst_hbm, ew_hbm, *rest):
        if pool:
            (batch_hbm, scale_hbm, shift_hbm, out_hbm, src_v, dst_v, ew_v,
             rows_v, acc, sem, chunk_v, bidx_v, scale_v, shift_v, pacc) = rest
        else:
            (out_hbm, src_v, dst_v, ew_v, rows_v, acc, sem) = rest
        c = lax.axis_index("c")
        s = lax.axis_index("s")
        cbase = c * HALF
        zero16 = jnp.zeros((16,), jnp.float32)

        def zrow(i, _):
            for u in range(8):
                r = i * 8 + u
                rows_v[r, 0:16] = zero16
                rows_v[r, 16:32] = zero16
            return 0

        lax.fori_loop(0, EC // 8, zrow, 0)

        def zacc(k, _):
            pltpu.sync_copy(rows_v.at[pl.ds(0, ZPT // 8)],
                            acc.at[pl.ds(s * ZPT + k * (ZPT // 8), ZPT // 8)])
            return 0

        lax.fori_loop(0, 8, zacc, 0)
        if pool:
            pltpu.sync_copy(rows_v.at[pl.ds(0, POOL_ROWS // NS)],
                            pacc.at[pl.ds(s * (POOL_ROWS // NS),
                                          POOL_ROWS // NS)])
            pltpu.sync_copy(scale_hbm, scale_v)
            pltpu.sync_copy(shift_hbm, shift_v)
        plsc.subcore_barrier()

        def chunk_body(k, _):
            chunk = s + k * NS

            @pl.when(chunk < NCHUNK_A)
            def _():
                pltpu.sync_copy(src_hbm.at[chunk], src_v)
                pltpu.sync_copy(dst_hbm.at[chunk], dst_v)
                pltpu.sync_copy(ew_hbm.at[pl.ds(chunk * EC, EC)], ew_v)
                cps = [
                    pltpu.async_copy(h_hbm.at[src_v.at[j]],
                                     rows_v.at[pl.ds(j * 128, 128)], sem)
                    for j in range(4)
                ]
                for cp in cps:
                    cp.wait()
                for j in range(4):
                    for i in range(8):
                        d = dst_v[j, pl.ds(i * 16, 16)]
                        dl = d - cbase
                        inr = (dl >= 0) & (dl < HALF)
                        tr = TRASH + (d & 255)
                        dst_v[j, pl.ds(i * 16, 16)] = jnp.where(inr, dl, tr)

                def wmul(g, _):
                    ew16 = ew_v[pl.ds(g * 16, 16)]
                    for k in range(16):
                        e = g * 16 + k
                        w = lax.gather(
                            ew16, jnp.full((16, 1), k, jnp.int32),
                            lax.GatherDimensionNumbers(
                                offset_dims=(), collapsed_slice_dims=(0,),
                                start_index_map=(0,)),
                            (1,),
                            mode=lax.GatherScatterMode.PROMISE_IN_BOUNDS)
                        rows_v[e, 0:16] = rows_v[e, 0:16] * w
                        rows_v[e, 16:32] = rows_v[e, 16:32] * w
                    return 0

                lax.fori_loop(0, EC // 16, wmul, 0)
                for j in range(4):
                    pltpu.sync_copy(rows_v.at[pl.ds(j * 128, 128)],
                                    acc.at[dst_v.at[j]], add=True)

            return 0

        lax.fori_loop(0, KA, chunk_body, 0)
        plsc.subcore_barrier()

        if not pool:
            nwb = HALF // 400

            def wb(k, _):
                wchunk = s + k * NS

                @pl.when(wchunk < nwb)
                def _():
                    pltpu.sync_copy(
                        acc.at[pl.ds(wchunk * 400, 400)],
                        out_hbm.at[pl.ds(c * HALF + wchunk * 400, 400)])

                return 0

            lax.fori_loop(0, (nwb + NS - 1) // NS, wb, 0)
            return

        sc_lo = scale_v[0:16]
        sc_hi = scale_v[16:32]
        sh_lo = shift_v[0:16]
        sh_hi = shift_v[16:32]

        def bchunk(k, _):
            chunk = s + k * NS

            @pl.when(chunk < NCHUNK_B)
            def _():
                pltpu.sync_copy(acc.at[pl.ds(chunk * CHB, CHB)], chunk_v)
                pltpu.sync_copy(batch_hbm.at[c, chunk], bidx_v)

                def rbody(i, _):
                    for u in range(4):
                        r = i * 4 + u
                        v0 = chunk_v[r, 0:16] * sc_lo + sh_lo
                        chunk_v[r, 0:16] = jnp.where(v0 >= 0, v0, 0.01 * v0)
                        v1 = chunk_v[r, 16:32] * sc_hi + sh_hi
                        chunk_v[r, 16:32] = jnp.where(v1 >= 0, v1, 0.01 * v1)
                    return 0

                lax.fori_loop(0, CHB // 4, rbody, 0)
                for j in range(2):
                    pltpu.sync_copy(chunk_v.at[pl.ds(j * 128, 128)],
                                    pacc.at[bidx_v.at[j]], add=True)

            return 0

        lax.fori_loop(0, KB, bchunk, 0)
        plsc.subcore_barrier()
        gpt = N_GRAPHS // NS
        pltpu.sync_copy(pacc.at[pl.ds(s * gpt, gpt)],
                        out_hbm.at[c, pl.ds(s * gpt, gpt)])

    return pl.kernel(
        body, out_type=out_type, mesh=_sc_mesh(), scratch_types=scratch,
        compiler_params=pltpu.CompilerParams(use_tc_tiling_on_sc=False))


_sc_layer1 = _make_sc_layer(pool=False)
_sc_layer2 = _make_sc_layer(pool=True)


def _tc_xw1(xp, w1p):
    def f(x_ref, w_ref, o_ref):
        o_ref[...] = jnp.dot(x_ref[...], w_ref[...],
                             preferred_element_type=jnp.float32)

    return pl.pallas_call(
        f,
        grid=(50,),
        in_specs=[pl.BlockSpec((2000, 8), lambda i: (i, 0)),
                  pl.BlockSpec((8, D), lambda i: (0, 0))],
        out_specs=pl.BlockSpec((2000, D), lambda i: (i, 0)),
        out_shape=jax.ShapeDtypeStruct((N_NODES, D), jnp.float32),
    )(xp, w1p)


def _tc_mid(s1, b1, w2p):
    def f(s_ref, b_ref, w_ref, o_ref):
        h = s_ref[...] + b_ref[...]
        h = jnp.where(h >= 0, h, 0.01 * h)
        o_ref[...] = jnp.dot(h, w_ref[...],
                             preferred_element_type=jnp.float32)

    return pl.pallas_call(
        f,
        grid=(50,),
        in_specs=[pl.BlockSpec((2000, D), lambda i: (i, 0)),
                  pl.BlockSpec((1, D), lambda i: (0, 0)),
                  pl.BlockSpec((D, D), lambda i: (0, 0))],
        out_specs=pl.BlockSpec((2000, D), lambda i: (i, 0)),
        out_shape=jax.ShapeDtypeStruct((N_NODES, D), jnp.float32),
    )(s1, b1.reshape(1, D), w2p)


def _tc_head(pp, fc1p, fc1b, fc2w, fc2b):
    def f(p_ref, w1_ref, b1_ref, w2_ref, b2_ref, o_ref):
        pooled = p_ref[0] + p_ref[1]
        z = jnp.dot(pooled, w1_ref[...],
                    preferred_element_type=jnp.float32) + b1_ref[...]
        z = jnp.where(z >= 0, z, 0.01 * z)
        o_ref[...] = jnp.dot(z, w2_ref[...],
                             preferred_element_type=jnp.float32) + b2_ref[...]

    return pl.pallas_call(
        f,
        out_shape=jax.ShapeDtypeStruct((N_GRAPHS, 2), jnp.float32),
    )(pp, fc1p, fc1b.reshape(1, 10), fc2w, fc2b.reshape(1, 2))


def kernel(x, edge_index, edge_weight, batch, W1, b1, W2, b2,
           bn_gamma, bn_beta, bn_mean, bn_var, fc1_w, fc1_b, fc2_w, fc2_b):
    src2 = edge_index[0].reshape(NCHUNK_A, 4, 128)
    dst2 = edge_index[1].reshape(NCHUNK_A, 4, 128)
    xp = jnp.pad(x, ((0, 0), (0, 2)))
    w1p = jnp.pad(W1, ((0, 2), (0, 0)))
    w2p = jnp.pad(W2, ((0, 0), (0, D - W2.shape[1])))
    scale = bn_gamma * jax.lax.rsqrt(bn_var + 1e-5)
    shift = (b2 - bn_mean) * scale + bn_beta
    scale_p = jnp.pad(scale, (0, D - scale.shape[0]))
    shift_p = jnp.pad(shift, (0, D - shift.shape[0]))
    trash_ids = N_GRAPHS + (jnp.arange(BPAD - HALF, dtype=jnp.int32) % 256)
    batch_ext = jnp.concatenate([batch, trash_ids])
    bslices = jnp.stack([batch_ext[0:BPAD],
                         batch_ext[HALF:HALF + BPAD]]).reshape(
                             NC, NCHUNK_B, 2, 128)
    fc1p = jnp.pad(fc1_w, ((0, D - fc1_w.shape[0]), (0, 0)))

    a = _tc_xw1(xp, w1p)
    s1 = _sc_layer1(a, src2, dst2, edge_weight)
    bmat = _tc_mid(s1, b1, w2p)
    pp = _sc_layer2(bmat, src2, dst2, edge_weight, bslices, scale_p, shift_p)
    return _tc_head(pp, fc1p, fc1_b, fc2_w, fc2_b)


# speedup vs baseline: 15.2683x; 1.7392x over previous
import functools

import jax
import jax.numpy as jnp
from jax import lax
from jax.experimental import pallas as pl
from jax.experimental.pallas import tpu as pltpu
from jax.experimental.pallas import tpu_sc as plsc

N_NODES = 100000
N_EDGES = 1600000
N_GRAPHS = 1024
D = 32

NC = 2
NS = 16

HALF = N_NODES // NC
CHB = 256
NCHUNK_B = 197
TRASH = NCHUNK_B * CHB
ACC_ROWS = TRASH + 256
NZC = ACC_ROWS // CHB
EC = 256
EB = EC // 128
KA = 396
NCHUNK_A = KA * NS
EPAD = NCHUNK_A * EC
KB = (NCHUNK_B + NS - 1) // NS
POOL_ROWS = N_GRAPHS + 64
BPAD = NCHUNK_B * CHB


def _sc_mesh():
    return plsc.VectorSubcoreMesh(core_axis_name="c", subcore_axis_name="s")


def _make_sc_layer(pool: bool):
    scratch = [
        pltpu.VMEM((2, EB, 128), jnp.int32),
        pltpu.VMEM((2, EB, 128), jnp.int32),
        pltpu.VMEM((2, EC), jnp.float32),
        pltpu.VMEM((3, EC, D), jnp.float32),
        pltpu.VMEM((3, EB, 128), jnp.int32),
        pltpu.VMEM_SHARED((ACC_ROWS, D), jnp.float32),
        pltpu.SemaphoreType.DMA,
        pltpu.SemaphoreType.DMA,
        pltpu.SemaphoreType.DMA,
        pltpu.SemaphoreType.DMA,
        pltpu.SemaphoreType.DMA,
        pltpu.SemaphoreType.DMA,
        pltpu.SemaphoreType.DMA,
        pltpu.SemaphoreType.DMA,
    ]
    if pool:
        scratch += [
            pltpu.VMEM((2, 128), jnp.int32),
            pltpu.VMEM((D,), jnp.float32),
            pltpu.VMEM((D,), jnp.float32),
            pltpu.VMEM_SHARED((POOL_ROWS, D), jnp.float32),
        ]
        out_type = jax.ShapeDtypeStruct((NC, N_GRAPHS, D), jnp.float32)
    else:
        out_type = jax.ShapeDtypeStruct((N_NODES, D), jnp.float32)

    dnums = lax.GatherDimensionNumbers(
        offset_dims=(), collapsed_slice_dims=(0,), start_index_map=(0,))

    def body(h_hbm, src_hbm, dst_hbm, ew_hbm, *rest):
        if pool:
            (batch_hbm, scale_hbm, shift_hbm, out_hbm, src_v, dst_v, ew_v,
             rows_v, dloc_v, acc, si0, si1, sg0, sg1, sg2, ss0, ss1, ss2,
             bidx_v, scale_v, shift_v, pacc) = rest
        else:
            (out_hbm, src_v, dst_v, ew_v, rows_v, dloc_v, acc,
             si0, si1, sg0, sg1, sg2, ss0, ss1, ss2) = rest
        sem_i = (si0, si1)
        sem_g = (sg0, sg1, sg2)
        sem_s = (ss0, ss1, ss2)
        c = lax.axis_index("c")
        s = lax.axis_index("s")
        cbase = c * HALF
        zero16 = jnp.zeros((16,), jnp.float32)

        def zrow(i, _):
            for u in range(8):
                r = i * 8 + u
                rows_v[0, r, 0:16] = zero16
                rows_v[0, r, 16:32] = zero16
            return 0

        lax.fori_loop(0, EC // 8, zrow, 0)

        def zacc(k, _):
            zc = s + k * NS

            @pl.when(zc < NZC)
            def _():
                pltpu.sync_copy(rows_v.at[0], acc.at[pl.ds(zc * CHB, CHB)])

            return 0

        lax.fori_loop(0, (NZC + NS - 1) // NS, zacc, 0)
        if pool:
            pltpu.sync_copy(rows_v.at[0, pl.ds(0, POOL_ROWS // NS)],
                            pacc.at[pl.ds(s * (POOL_ROWS // NS),
                                          POOL_ROWS // NS)])
            pltpu.sync_copy(scale_hbm, scale_v)
            pltpu.sync_copy(shift_hbm, shift_v)
        plsc.subcore_barrier()

        def issue_idx(kk, sl):
            ch = s + kk * NS
            pltpu.async_copy(src_hbm.at[ch], src_v.at[sl], sem_i[sl])
            pltpu.async_copy(dst_hbm.at[ch], dst_v.at[sl], sem_i[sl])
            pltpu.async_copy(ew_hbm.at[pl.ds(ch * EC, EC)], ew_v.at[sl],
                             sem_i[sl])

        def wait_idx(sl):
            pltpu.make_async_copy(src_hbm.at[0], src_v.at[sl],
                                  sem_i[sl]).wait()
            pltpu.make_async_copy(dst_hbm.at[0], dst_v.at[sl],
                                  sem_i[sl]).wait()
            pltpu.make_async_copy(ew_hbm.at[pl.ds(0, EC)], ew_v.at[sl],
                                  sem_i[sl]).wait()

        def issue_gather(sl2, sl3):
            for j in range(EB):
                pltpu.async_copy(h_hbm.at[src_v.at[sl2, j]],
                                 rows_v.at[sl3, pl.ds(j * 128, 128)],
                                 sem_g[sl3])

        def wait_rows(sem):
            pltpu.make_async_copy(h_hbm.at[pl.ds(0, EC)], rows_v.at[0],
                                  sem).wait()

        def issue_scatter(sl3):
            for j in range(EB):
                pltpu.async_copy(rows_v.at[sl3, pl.ds(j * 128, 128)],
                                 acc.at[dloc_v.at[sl3, j]], sem_s[sl3],
                                 add=True)

        def localize(sl2, sl3):
            for j in range(EB):
                for i in range(8):
                    d = dst_v[sl2, j, pl.ds(i * 16, 16)]
                    dl = d - cbase
                    inr = (dl >= 0) & (dl < HALF)
                    tr = TRASH + (d & 255)
                    dloc_v[sl3, j, pl.ds(i * 16, 16)] = jnp.where(inr, dl, tr)

        def wmul(sl2, sl3):
            def g_body(g, _):
                ew16 = ew_v[sl2, pl.ds(g * 16, 16)]
                for u in range(16):
                    e = g * 16 + u
                    w = lax.gather(
                        ew16, jnp.full((16, 1), u, jnp.int32), dnums, (1,),
                        mode=lax.GatherScatterMode.PROMISE_IN_BOUNDS)
                    rows_v[sl3, e, 0:16] = rows_v[sl3, e, 0:16] * w
                    rows_v[sl3, e, 16:32] = rows_v[sl3, e, 16:32] * w
                return 0

            lax.fori_loop(0, EC // 16, g_body, 0)

        issue_idx(0, 0)
        issue_idx(1, 1)
        wait_idx(0)
        issue_gather(0, 0)

        def pipe(k6, _):
            for j in range(6):
                kk = k6 * 6 + j
                sl2, o2 = j % 2, (j + 1) % 2
                sl3, n3 = j % 3, (j + 1) % 3

                @pl.when(kk + 1 < KA)
                def _():
                    wait_idx(o2)

                @pl.when(kk >= 2)
                def _():
                    wait_rows(sem_s[n3])

                @pl.when(kk + 1 < KA)
                def _():
                    issue_gather(o2, n3)

                wait_rows(sem_g[sl3])
                localize(sl2, sl3)
                wmul(sl2, sl3)

                @pl.when(kk + 2 < KA)
                def _():
                    issue_idx(kk + 2, sl2)

                issue_scatter(sl3)
            return 0

        lax.fori_loop(0, KA // 6, pipe, 0)
        wait_rows(sem_s[(KA - 2) % 3])
        wait_rows(sem_s[(KA - 1) % 3])
        plsc.subcore_barrier()

        if not pool:
            nwb = HALF // 400

            def wb(k, _):
                wchunk = s + k * NS

                @pl.when(wchunk < nwb)
                def _():
                    pltpu.sync_copy(
                        acc.at[pl.ds(wchunk * 400, 400)],
                        out_hbm.at[pl.ds(c * HALF + wchunk * 400, 400)])

                return 0

            lax.fori_loop(0, (nwb + NS - 1) // NS, wb, 0)
            return

        sc_lo = scale_v[0:16]
        sc_hi = scale_v[16:32]
        sh_lo = shift_v[0:16]
        sh_hi = shift_v[16:32]

        def bchunk(k, _):
            chunk = s + k * NS

            @pl.when(chunk < NCHUNK_B)
            def _():
                pltpu.sync_copy(acc.at[pl.ds(chunk * CHB, CHB)],
                                rows_v.at[0])
                pltpu.sync_copy(batch_hbm.at[c, chunk], bidx_v)

                def rbody(i, _):
                    for u in range(4):
                        r = i * 4 + u
                        v0 = rows_v[0, r, 0:16] * sc_lo + sh_lo
                        rows_v[0, r, 0:16] = jnp.where(v0 >= 0, v0, 0.01 * v0)
                        v1 = rows_v[0, r, 16:32] * sc_hi + sh_hi
                        rows_v[0, r, 16:32] = jnp.where(v1 >= 0, v1,
                                                        0.01 * v1)
                    return 0

                lax.fori_loop(0, CHB // 4, rbody, 0)
                for j in range(2):
                    pltpu.sync_copy(rows_v.at[0, pl.ds(j * 128, 128)],
                                    pacc.at[bidx_v.at[j]], add=True)

            return 0

        lax.fori_loop(0, KB, bchunk, 0)
        plsc.subcore_barrier()
        gpt = N_GRAPHS // NS
        pltpu.sync_copy(pacc.at[pl.ds(s * gpt, gpt)],
                        out_hbm.at[c, pl.ds(s * gpt, gpt)])

    return pl.kernel(
        body, out_type=out_type, mesh=_sc_mesh(), scratch_types=scratch,
        compiler_params=pltpu.CompilerParams(use_tc_tiling_on_sc=False))


_sc_layer1 = _make_sc_layer(pool=False)
_sc_layer2 = _make_sc_layer(pool=True)


def _tc_xw1(xp, w1p):
    def f(x_ref, w_ref, o_ref):
        o_ref[...] = jnp.dot(x_ref[...], w_ref[...],
                             preferred_element_type=jnp.float32)

    return pl.pallas_call(
        f,
        grid=(50,),
        in_specs=[pl.BlockSpec((2000, 8), lambda i: (i, 0)),
                  pl.BlockSpec((8, D), lambda i: (0, 0))],
        out_specs=pl.BlockSpec((2000, D), lambda i: (i, 0)),
        out_shape=jax.ShapeDtypeStruct((N_NODES, D), jnp.float32),
    )(xp, w1p)


def _tc_mid(s1, b1, w2p):
    def f(s_ref, b_ref, w_ref, o_ref):
        h = s_ref[...] + b_ref[...]
        h = jnp.where(h >= 0, h, 0.01 * h)
        o_ref[...] = jnp.dot(h, w_ref[...],
                             preferred_element_type=jnp.float32)

    return pl.pallas_call(
        f,
        grid=(50,),
        in_specs=[pl.BlockSpec((2000, D), lambda i: (i, 0)),
                  pl.BlockSpec((1, D), lambda i: (0, 0)),
                  pl.BlockSpec((D, D), lambda i: (0, 0))],
        out_specs=pl.BlockSpec((2000, D), lambda i: (i, 0)),
        out_shape=jax.ShapeDtypeStruct((N_NODES, D), jnp.float32),
    )(s1, b1.reshape(1, D), w2p)


def _tc_head(pp, fc1p, fc1b, fc2w, fc2b):
    def f(p_ref, w1_ref, b1_ref, w2_ref, b2_ref, o_ref):
        pooled = p_ref[0] + p_ref[1]
        z = jnp.dot(pooled, w1_ref[...],
                    preferred_element_type=jnp.float32) + b1_ref[...]
        z = jnp.where(z >= 0, z, 0.01 * z)
        o_ref[...] = jnp.dot(z, w2_ref[...],
                             preferred_element_type=jnp.float32) + b2_ref[...]

    return pl.pallas_call(
        f,
        out_shape=jax.ShapeDtypeStruct((N_GRAPHS, 2), jnp.float32),
    )(pp, fc1p, fc1b.reshape(1, 10), fc2w, fc2b.reshape(1, 2))


def kernel(x, edge_index, edge_weight, batch, W1, b1, W2, b2,
           bn_gamma, bn_beta, bn_mean, bn_var, fc1_w, fc1_b, fc2_w, fc2_b):
    npad = EPAD - N_EDGES
    ar = jnp.arange(npad, dtype=jnp.int32)
    src2 = jnp.concatenate([edge_index[0], ar % N_NODES]).reshape(
        NCHUNK_A, EB, 128)
    dst2 = jnp.concatenate([edge_index[1], N_NODES + (ar % 256)]).reshape(
        NCHUNK_A, EB, 128)
    ew_full = jnp.concatenate(
        [edge_weight, jnp.zeros((npad,), jnp.float32)])
    xp = jnp.pad(x, ((0, 0), (0, 2)))
    w1p = jnp.pad(W1, ((0, 2), (0, 0)))
    w2p = jnp.pad(W2, ((0, 0), (0, D - W2.shape[1])))
    scale = bn_gamma * jax.lax.rsqrt(bn_var + 1e-5)
    shift = (b2 - bn_mean) * scale + bn_beta
    scale_p = jnp.pad(scale, (0, D - scale.shape[0]))
    shift_p = jnp.pad(shift, (0, D - shift.shape[0]))
    trash_ids = N_GRAPHS + (jnp.arange(BPAD - HALF, dtype=jnp.int32) % 64)
    batch_ext = jnp.concatenate([batch, trash_ids])
    bslices = jnp.stack([batch_ext[0:BPAD],
                         batch_ext[HALF:HALF + BPAD]]).reshape(
                             NC, NCHUNK_B, 2, 128)
    fc1p = jnp.pad(fc1_w, ((0, D - fc1_w.shape[0]), (0, 0)))

    a = _tc_xw1(xp, w1p)
    s1 = _sc_layer1(a, src2, dst2, ew_full)
    bmat = _tc_mid(s1, b1, w2p)
    pp = _sc_layer2(bmat, src2, dst2, ew_full, bslices, scale_p, shift_p)
    return _tc_head(pp, fc1p, fc1_b, fc2_w, fc2_b)
